# baseline (speedup 1.0000x reference)
import jax
import jax.numpy as jnp
from jax.experimental import pallas as pl

K = 8192
D = 4
N = 8192
TOK = 512
NB = N // TOK
BETA = 0.25


def _vq_body(z_ref, cbt_ref, idx_ref, sse_ref):
    i = pl.program_id(0)
    zb = z_ref[...]
    cbt = cbt_ref[...]
    prod = jax.lax.dot_general(zb.astype(jnp.bfloat16), cbt.astype(jnp.bfloat16),
                               (((1,), (0,)), ((), ())),
                               preferred_element_type=jnp.float32)
    z2 = jnp.sum(zb * zb, axis=1, keepdims=True)
    c2 = jnp.sum(cbt * cbt, axis=0, keepdims=True)
    dist = jnp.sqrt(jnp.maximum((z2 + c2) - 2.0 * prod, 0.0))
    dmin = jnp.min(dist, axis=1, keepdims=True)
    kiota = jax.lax.broadcasted_iota(jnp.int32, (TOK, K), 1)
    idx = jnp.min(jnp.where(dist == dmin, kiota, K), axis=1).astype(jnp.int32)
    idx_ref[0, 0, :] = idx
    part = jnp.sum(dmin * dmin)

    @pl.when(i == 0)
    def _():
        sse_ref[...] = jnp.zeros((1, 1), jnp.float32)

    sse_ref[...] = sse_ref[...] + part

    @pl.when(i == NB - 1)
    def _():
        sse_ref[...] = sse_ref[...] * ((1.0 + BETA) / (N * D))


def _vq_quantize(z_flat, cbt):
    return pl.pallas_call(
        _vq_body,
        grid=(NB,),
        in_specs=[
            pl.BlockSpec((TOK, D), lambda i: (i, 0)),
            pl.BlockSpec((D, K), lambda i: (0, 0)),
        ],
        out_specs=[
            pl.BlockSpec((1, 1, TOK), lambda i: (i, 0, 0)),
            pl.BlockSpec((1, 1), lambda i: (0, 0)),
        ],
        out_shape=[
            jax.ShapeDtypeStruct((NB, 1, TOK), jnp.int32),
            jax.ShapeDtypeStruct((1, 1), jnp.float32),
        ],
    )(z_flat, cbt)


def kernel(z_e, codebook):
    B, C, H, W = z_e.shape
    z_flat = jnp.transpose(z_e, (0, 2, 3, 1)).reshape(-1, D)
    idx3, sse = _vq_quantize(z_flat, codebook.T)
    indices = idx3.reshape(-1)
    z_q_flat = jnp.take(codebook, indices, axis=0)
    z_q = jnp.transpose(z_q_flat.reshape(B, H, W, D), (0, 3, 1, 2))
    z_q_st = z_e + (z_q - z_e)
    total_loss = sse[0, 0]
    return (total_loss, z_q_st, indices.reshape(B, H, W))


# speedup vs baseline: 1.4816x; 1.4816x over previous
import jax
import jax.numpy as jnp
from jax.experimental import pallas as pl

K = 8192
D = 4
N = 8192
TOK = 512
NB = N // TOK
BETA = 0.25


def _vq_body(z_ref, cbt_ref, idx_ref, sse_ref):
    i = pl.program_id(0)
    zb = z_ref[...]
    cbt = cbt_ref[...]
    prod = jax.lax.dot_general(zb.astype(jnp.bfloat16), cbt.astype(jnp.bfloat16),
                               (((1,), (0,)), ((), ())),
                               preferred_element_type=jnp.float32)
    z2 = jnp.sum(zb * zb, axis=1, keepdims=True)
    c2 = jnp.sum(cbt * cbt, axis=0, keepdims=True)
    d2 = jnp.maximum((z2 + c2) - 2.0 * prod, 0.0)
    dmin = jnp.min(d2, axis=1)
    idx = jnp.argmin(d2, axis=1).astype(jnp.int32)
    idx_ref[0, 0, :] = idx
    part = jnp.sum(dmin)

    @pl.when(i == 0)
    def _():
        sse_ref[...] = jnp.zeros((1, 1), jnp.float32)

    sse_ref[...] = sse_ref[...] + part

    @pl.when(i == NB - 1)
    def _():
        sse_ref[...] = sse_ref[...] * ((1.0 + BETA) / (N * D))


def _vq_quantize(z_flat, cbt):
    return pl.pallas_call(
        _vq_body,
        grid=(NB,),
        in_specs=[
            pl.BlockSpec((TOK, D), lambda i: (i, 0)),
            pl.BlockSpec((D, K), lambda i: (0, 0)),
        ],
        out_specs=[
            pl.BlockSpec((1, 1, TOK), lambda i: (i, 0, 0)),
            pl.BlockSpec((1, 1), lambda i: (0, 0)),
        ],
        out_shape=[
            jax.ShapeDtypeStruct((NB, 1, TOK), jnp.int32),
            jax.ShapeDtypeStruct((1, 1), jnp.float32),
        ],
    )(z_flat, cbt)


def kernel(z_e, codebook):
    B, C, H, W = z_e.shape
    z_flat = jnp.transpose(z_e, (0, 2, 3, 1)).reshape(-1, D)
    idx3, sse = _vq_quantize(z_flat, codebook.T)
    indices = idx3.reshape(-1)
    z_q_flat = jnp.take(codebook, indices, axis=0)
    z_q = jnp.transpose(z_q_flat.reshape(B, H, W, D), (0, 3, 1, 2))
    z_q_st = z_e + (z_q - z_e)
    total_loss = sse[0, 0]
    return (total_loss, z_q_st, indices.reshape(B, H, W))
